# baseline (speedup 1.0000x reference)
import functools

import jax
import jax.numpy as jnp
from jax import lax
from jax.experimental import pallas as pl
from jax.experimental.pallas import tpu as pltpu
from jax.experimental.pallas import tpu_sc as plsc

C = 128
N = 10000
K = 32
NUM_CORES = 2
NUM_SUBCORES = 16
NUM_WORKERS = NUM_CORES * NUM_SUBCORES
N_PAD = 10240
PER_WORKER = N_PAD // NUM_WORKERS
NB = 64
NCHUNKS = PER_WORKER // NB
LANES = 16

STAGE_ROWS = 632
STAGE_LAST = N - 15 * STAGE_ROWS


def _sc_gather_sum(xt, idx_w, eps_v):
  mesh = plsc.VectorSubcoreMesh(
      core_axis_name="c", subcore_axis_name="s")

  @functools.partial(
      pl.kernel,
      mesh=mesh,
      out_type=jax.ShapeDtypeStruct((N_PAD, C), jnp.float32),
      scratch_types=[
          pltpu.VMEM((NCHUNKS * K, NB), jnp.int32),
          pltpu.VMEM((NB, C), jnp.float32),
          pltpu.VMEM((NB, C), jnp.float32),
          pltpu.VMEM((NB, C), jnp.float32),
          pltpu.VMEM((LANES,), jnp.float32),
          pltpu.VMEM_SHARED((N_PAD, C), jnp.float32),
          pltpu.SemaphoreType.DMA,
          pltpu.SemaphoreType.DMA,
          pltpu.SemaphoreType.DMA,
      ],
  )
  def body(xt_hbm, idxw_hbm, eps_hbm, out_hbm, idx_t, acc0, acc1, xb,
           eps_s, tbl_s, sem_a, sem_b, sem_x):
    sid = lax.axis_index("s")
    wid = sid * NUM_CORES + lax.axis_index("c")
    base = wid * PER_WORKER

    @pl.when(sid < 15)
    def _stage_main():
      off = pl.multiple_of(sid * STAGE_ROWS, 8)
      pltpu.sync_copy(xt_hbm.at[pl.ds(off, STAGE_ROWS)],
                      tbl_s.at[pl.ds(off, STAGE_ROWS)])

    @pl.when(sid == 15)
    def _stage_last():
      pltpu.sync_copy(xt_hbm.at[pl.ds(15 * STAGE_ROWS, STAGE_LAST)],
                      tbl_s.at[pl.ds(15 * STAGE_ROWS, STAGE_LAST)])

    pltpu.sync_copy(idxw_hbm.at[wid], idx_t)
    pltpu.sync_copy(eps_hbm, eps_s)

    plsc.subcore_barrier()

    zv = jnp.zeros((LANES,), jnp.float32)
    scale = eps_s[pl.ds(0, LANES)] + 1.0
    accs = (acc0, acc1)
    sems = (sem_a, sem_b)

    def zero(acc):
      @pl.loop(0, NB)
      def _z(r):
        for cs in range(C // LANES):
          acc[r, pl.ds(cs * LANES, LANES)] = zv

    def fire(c, acc, sem):
      @pl.loop(0, K)
      def _f(k):
        pltpu.async_copy(tbl_s.at[idx_t.at[c * K + k]], acc, sem, add=True)

    def drain(acc, sem):
      @pl.loop(0, K)
      def _d(k):
        pltpu.make_async_copy(tbl_s.at[idx_t.at[0]], acc, sem).wait()

    def fetch_self(c):
      pltpu.async_copy(tbl_s.at[pl.ds(base + c * NB, NB)], xb, sem_x)

    def self_fma(acc):
      pltpu.make_async_copy(tbl_s.at[pl.ds(base, NB)], xb, sem_x).wait()

      @pl.loop(0, NB)
      def _sf(r):
        for cs in range(C // LANES):
          sl = pl.ds(cs * LANES, LANES)
          acc[r, sl] = acc[r, sl] + scale * xb[r, sl]

    fetch_self(0)
    for c in range(NCHUNKS):
      b, ob = c % 2, (c - 1) % 2
      zero(accs[b])
      fire(c, accs[b], sems[b])
      if c > 0:
        drain(accs[ob], sems[ob])
        self_fma(accs[ob])
        fetch_self(c)
        pltpu.sync_copy(accs[ob], out_hbm.at[pl.ds(base + (c - 1) * NB, NB)])
    last = NCHUNKS - 1
    drain(accs[last % 2], sems[last % 2])
    self_fma(accs[last % 2])
    pltpu.sync_copy(accs[last % 2],
                    out_hbm.at[pl.ds(base + last * NB, NB)])

  return body(xt, idx_w, eps_v)


BN = 512


def _tc_out(s, w, b1d):
  def body(w_ref, b_ref, s_ref, o_ref):
    t = lax.dot_general(
        s_ref[...], w_ref[...],
        dimension_numbers=(((1,), (1,)), ((), ())),
        preferred_element_type=jnp.float32,
    )
    o_ref[...] = jnp.maximum(t + b_ref[...], 0.0)

  grid = (pl.cdiv(N, BN),)
  return pl.pallas_call(
      body,
      grid=grid,
      in_specs=[
          pl.BlockSpec((C, C), lambda i: (0, 0)),
          pl.BlockSpec((1, C), lambda i: (0, 0)),
          pl.BlockSpec((BN, C), lambda i: (i, 0)),
      ],
      out_specs=pl.BlockSpec((BN, C), lambda i: (i, 0)),
      out_shape=jax.ShapeDtypeStruct((N, C), jnp.float32),
  )(w, b1d, s)


def kernel(x, edge_index, W, b, eps):
  x2d = x.reshape(C, N)
  xt = x2d.T
  idx = edge_index[0].reshape(N, K)
  pad_idx = (jnp.arange((N_PAD - N) * K, dtype=jnp.int32) % N).reshape(
      N_PAD - N, K)
  idx_w = (
      jnp.concatenate([idx, pad_idx], axis=0)
      .reshape(NUM_WORKERS, NCHUNKS, NB, K)
      .transpose(0, 1, 3, 2)
      .reshape(NUM_WORKERS, NCHUNKS * K, NB)
  )
  eps_v = jnp.broadcast_to(
      eps.reshape(-1)[:1].astype(jnp.float32), (LANES,))

  s = _sc_gather_sum(xt, idx_w, eps_v)

  b1d = b.reshape(1, C)
  out_t = _tc_out(s, W, b1d)
  return out_t.T.reshape(1, C, N, 1)


# speedup vs baseline: 1.0381x; 1.0381x over previous
import functools

import jax
import jax.numpy as jnp
from jax import lax
from jax.experimental import pallas as pl
from jax.experimental.pallas import tpu as pltpu
from jax.experimental.pallas import tpu_sc as plsc

C = 128
N = 10000
K = 32
NUM_CORES = 2
NUM_SUBCORES = 16
NUM_WORKERS = NUM_CORES * NUM_SUBCORES
N_PAD = 10240
PER_WORKER = N_PAD // NUM_WORKERS
NB = 64
NCHUNKS = PER_WORKER // NB
LANES = 16

STAGE_ROWS = 632
STAGE_LAST = N - 15 * STAGE_ROWS


def _sc_gather_sum(xt, idx_w):
  mesh = plsc.VectorSubcoreMesh(
      core_axis_name="c", subcore_axis_name="s")

  @functools.partial(
      pl.kernel,
      mesh=mesh,
      out_type=jax.ShapeDtypeStruct((N_PAD, C), jnp.float32),
      scratch_types=[
          pltpu.VMEM((NCHUNKS * K, NB), jnp.int32),
          pltpu.VMEM((NB, C), jnp.float32),
          pltpu.VMEM((NB, C), jnp.float32),
          pltpu.VMEM_SHARED((N, C), jnp.float32),
          pltpu.SemaphoreType.DMA,
          pltpu.SemaphoreType.DMA,
      ],
  )
  def body(xt_hbm, idxw_hbm, out_hbm, idx_t, acc0, acc1,
           tbl_s, sem_a, sem_b):
    sid = lax.axis_index("s")
    wid = sid * NUM_CORES + lax.axis_index("c")
    base = wid * PER_WORKER

    @pl.when(sid < 15)
    def _stage_main():
      off = pl.multiple_of(sid * STAGE_ROWS, 8)
      pltpu.sync_copy(xt_hbm.at[pl.ds(off, STAGE_ROWS)],
                      tbl_s.at[pl.ds(off, STAGE_ROWS)])

    @pl.when(sid == 15)
    def _stage_last():
      pltpu.sync_copy(xt_hbm.at[pl.ds(15 * STAGE_ROWS, STAGE_LAST)],
                      tbl_s.at[pl.ds(15 * STAGE_ROWS, STAGE_LAST)])

    pltpu.sync_copy(idxw_hbm.at[wid], idx_t)

    plsc.subcore_barrier()

    zv = jnp.zeros((LANES,), jnp.float32)
    accs = (acc0, acc1)
    sems = (sem_a, sem_b)

    def zero(acc):
      @pl.loop(0, NB)
      def _z(r):
        for cs in range(C // LANES):
          acc[r, pl.ds(cs * LANES, LANES)] = zv

    def fire(c, acc, sem):
      @pl.loop(0, K)
      def _f(k):
        pltpu.async_copy(tbl_s.at[idx_t.at[c * K + k]], acc, sem, add=True)

    def drain(acc, sem):
      @pl.loop(0, K)
      def _d(k):
        pltpu.make_async_copy(tbl_s.at[idx_t.at[0]], acc, sem).wait()

    for c in range(NCHUNKS):
      b, ob = c % 2, (c - 1) % 2
      zero(accs[b])
      fire(c, accs[b], sems[b])
      if c > 0:
        drain(accs[ob], sems[ob])
        pltpu.sync_copy(accs[ob], out_hbm.at[pl.ds(base + (c - 1) * NB, NB)])
    last = NCHUNKS - 1
    drain(accs[last % 2], sems[last % 2])
    pltpu.sync_copy(accs[last % 2],
                    out_hbm.at[pl.ds(base + last * NB, NB)])

  return body(xt, idx_w)


BN = 512


def _tc_self(x2d, w, b1d, eps2d):
  def body(eps_ref, w_ref, b_ref, x_ref, o_ref):
    scale = 1.0 + eps_ref[0, 0]
    o_ref[...] = lax.dot_general(
        x_ref[...] * scale, w_ref[...],
        dimension_numbers=(((0,), (1,)), ((), ())),
        preferred_element_type=jnp.float32,
    ) + b_ref[...]

  grid = (pl.cdiv(N, BN),)
  return pl.pallas_call(
      body,
      grid=grid,
      in_specs=[
          pl.BlockSpec((1, 1), lambda i: (0, 0)),
          pl.BlockSpec((C, C), lambda i: (0, 0)),
          pl.BlockSpec((1, C), lambda i: (0, 0)),
          pl.BlockSpec((C, BN), lambda i: (0, i)),
      ],
      out_specs=pl.BlockSpec((BN, C), lambda i: (i, 0)),
      out_shape=jax.ShapeDtypeStruct((N, C), jnp.float32),
  )(eps2d, w, b1d, x2d)


def _tc_neigh(t1t, s, w):
  def body(w_ref, t1_ref, s_ref, o_ref):
    t2 = lax.dot_general(
        s_ref[...].astype(jnp.bfloat16), w_ref[...].astype(jnp.bfloat16),
        dimension_numbers=(((1,), (1,)), ((), ())),
        preferred_element_type=jnp.float32,
    )
    o_ref[...] = jnp.maximum(t1_ref[...] + t2, 0.0)

  grid = (pl.cdiv(N, BN),)
  return pl.pallas_call(
      body,
      grid=grid,
      in_specs=[
          pl.BlockSpec((C, C), lambda i: (0, 0)),
          pl.BlockSpec((BN, C), lambda i: (i, 0)),
          pl.BlockSpec((BN, C), lambda i: (i, 0)),
      ],
      out_specs=pl.BlockSpec((BN, C), lambda i: (i, 0)),
      out_shape=jax.ShapeDtypeStruct((N, C), jnp.float32),
  )(w, t1t, s)


def kernel(x, edge_index, W, b, eps):
  x2d = x.reshape(C, N)
  xt = x2d.T
  idx = edge_index[0].reshape(N, K)
  pad_idx = (jnp.arange((N_PAD - N) * K, dtype=jnp.int32) % N).reshape(
      N_PAD - N, K)
  idx_w = (
      jnp.concatenate([idx, pad_idx], axis=0)
      .reshape(NUM_WORKERS, NCHUNKS, NB, K)
      .transpose(0, 1, 3, 2)
      .reshape(NUM_WORKERS, NCHUNKS * K, NB)
  )
  s = _sc_gather_sum(xt, idx_w)

  b1d = b.reshape(1, C)
  eps2d = eps.reshape(1, 1)
  t1t = _tc_self(x2d, W, b1d, eps2d)
  out_t = _tc_neigh(t1t, s, W)
  return out_t.T.reshape(1, C, N, 1)


# speedup vs baseline: 1.0903x; 1.0503x over previous
import functools

import jax
import jax.numpy as jnp
from jax import lax
from jax.experimental import pallas as pl
from jax.experimental.pallas import tpu as pltpu
from jax.experimental.pallas import tpu_sc as plsc

C = 128
N = 10000
K = 32
NUM_CORES = 2
NUM_SUBCORES = 16
NUM_WORKERS = NUM_CORES * NUM_SUBCORES
N_PAD = 10240
PER_WORKER = N_PAD // NUM_WORKERS
NB = 64
NCHUNKS = PER_WORKER // NB
LANES = 16

STAGE_ROWS = 632
STAGE_LAST = N - 15 * STAGE_ROWS


def _sc_gather_sum(xt, idx_w):
  mesh = plsc.VectorSubcoreMesh(
      core_axis_name="c", subcore_axis_name="s")

  @functools.partial(
      pl.kernel,
      mesh=mesh,
      out_type=jax.ShapeDtypeStruct((N_PAD, C), jnp.float32),
      scratch_types=[
          pltpu.VMEM((NCHUNKS * K, NB), jnp.int32),
          pltpu.VMEM((NB, C), jnp.float32),
          pltpu.VMEM((NB, C), jnp.float32),
          pltpu.VMEM_SHARED((N, C), jnp.float32),
          pltpu.SemaphoreType.DMA,
          pltpu.SemaphoreType.DMA,
      ],
  )
  def body(xt_hbm, idxw_hbm, out_hbm, idx_t, acc0, acc1,
           tbl_s, sem_a, sem_b):
    sid = lax.axis_index("s")
    wid = sid * NUM_CORES + lax.axis_index("c")
    base = wid * PER_WORKER

    @pl.when(sid < 15)
    def _stage_main():
      off = pl.multiple_of(sid * STAGE_ROWS, 8)
      pltpu.sync_copy(xt_hbm.at[pl.ds(off, STAGE_ROWS)],
                      tbl_s.at[pl.ds(off, STAGE_ROWS)])

    @pl.when(sid == 15)
    def _stage_last():
      pltpu.sync_copy(xt_hbm.at[pl.ds(15 * STAGE_ROWS, STAGE_LAST)],
                      tbl_s.at[pl.ds(15 * STAGE_ROWS, STAGE_LAST)])

    pltpu.sync_copy(idxw_hbm.at[wid], idx_t)

    plsc.subcore_barrier()

    zv = jnp.zeros((LANES,), jnp.float32)
    accs = (acc0, acc1)
    sems = (sem_a, sem_b)

    def zero(acc):
      @pl.loop(0, NB)
      def _z(r):
        for cs in range(C // LANES):
          acc[r, pl.ds(cs * LANES, LANES)] = zv

    def fire(c, acc, sem):
      @pl.loop(0, K)
      def _f(k):
        pltpu.async_copy(tbl_s.at[idx_t.at[c * K + k]], acc, sem, add=True)

    def drain(acc, sem):
      @pl.loop(0, K)
      def _d(k):
        pltpu.make_async_copy(tbl_s.at[idx_t.at[0]], acc, sem).wait()

    for c in range(NCHUNKS):
      b, ob = c % 2, (c - 1) % 2
      zero(accs[b])
      fire(c, accs[b], sems[b])
      if c > 0:
        drain(accs[ob], sems[ob])
        pltpu.sync_copy(accs[ob], out_hbm.at[pl.ds(base + (c - 1) * NB, NB)])
    last = NCHUNKS - 1
    drain(accs[last % 2], sems[last % 2])
    pltpu.sync_copy(accs[last % 2],
                    out_hbm.at[pl.ds(base + last * NB, NB)])

  return body(xt, idx_w)


BN = 1024


def _tc_self(x2d, w, b1d, eps2d):
  def body(eps_ref, w_ref, b_ref, x_ref, o_ref):
    scale = 1.0 + eps_ref[0, 0]
    o_ref[...] = lax.dot_general(
        x_ref[...] * scale, w_ref[...],
        dimension_numbers=(((0,), (1,)), ((), ())),
        preferred_element_type=jnp.float32,
    ) + b_ref[...]

  grid = (pl.cdiv(N, BN),)
  return pl.pallas_call(
      body,
      grid=grid,
      in_specs=[
          pl.BlockSpec((1, 1), lambda i: (0, 0)),
          pl.BlockSpec((C, C), lambda i: (0, 0)),
          pl.BlockSpec((1, C), lambda i: (0, 0)),
          pl.BlockSpec((C, BN), lambda i: (0, i)),
      ],
      out_specs=pl.BlockSpec((BN, C), lambda i: (i, 0)),
      out_shape=jax.ShapeDtypeStruct((N, C), jnp.float32),
  )(eps2d, w, b1d, x2d)


def _tc_neigh(t1t, s, w):
  def body(w_ref, t1_ref, s_ref, o_ref):
    t2 = lax.dot_general(
        s_ref[...], w_ref[...],
        dimension_numbers=(((1,), (1,)), ((), ())),
        preferred_element_type=jnp.float32,
    )
    o_ref[...] = jnp.maximum(t1_ref[...] + t2, 0.0)

  grid = (pl.cdiv(N, BN),)
  return pl.pallas_call(
      body,
      grid=grid,
      in_specs=[
          pl.BlockSpec((C, C), lambda i: (0, 0)),
          pl.BlockSpec((BN, C), lambda i: (i, 0)),
          pl.BlockSpec((BN, C), lambda i: (i, 0)),
      ],
      out_specs=pl.BlockSpec((BN, C), lambda i: (i, 0)),
      out_shape=jax.ShapeDtypeStruct((N, C), jnp.float32),
  )(w, t1t, s)


def kernel(x, edge_index, W, b, eps):
  x2d = x.reshape(C, N)
  xt = x2d.T
  idx = edge_index[0].reshape(N, K)
  pad_idx = (jnp.arange((N_PAD - N) * K, dtype=jnp.int32) % N).reshape(
      N_PAD - N, K)
  idx_w = (
      jnp.concatenate([idx, pad_idx], axis=0)
      .reshape(NUM_WORKERS, NCHUNKS, NB, K)
      .transpose(0, 1, 3, 2)
      .reshape(NUM_WORKERS, NCHUNKS * K, NB)
  )
  s = _sc_gather_sum(xt, idx_w)

  b1d = b.reshape(1, C)
  eps2d = eps.reshape(1, 1)
  t1t = _tc_self(x2d, W, b1d, eps2d)
  out_t = _tc_neigh(t1t, s, W)
  return out_t.T.reshape(1, C, N, 1)


# speedup vs baseline: 1.1150x; 1.0227x over previous
import functools

import jax
import jax.numpy as jnp
from jax import lax
from jax.experimental import pallas as pl
from jax.experimental.pallas import tpu as pltpu
from jax.experimental.pallas import tpu_sc as plsc

C = 128
N = 10000
K = 32
NUM_CORES = 2
NUM_SUBCORES = 16
NUM_WORKERS = NUM_CORES * NUM_SUBCORES
N_PAD = 10240
PER_WORKER = N_PAD // NUM_WORKERS
NB = 64
NCHUNKS = PER_WORKER // NB
LANES = 16

STAGE_ROWS = 632
STAGE_LAST = N - 15 * STAGE_ROWS


def _sc_gather_sum(xt, idx_w):
  mesh = plsc.VectorSubcoreMesh(
      core_axis_name="c", subcore_axis_name="s")

  @functools.partial(
      pl.kernel,
      mesh=mesh,
      out_type=jax.ShapeDtypeStruct((N_PAD, C), jnp.float32),
      scratch_types=[
          pltpu.VMEM((NCHUNKS * K, NB), jnp.int32),
          pltpu.VMEM((NB, C), jnp.float32),
          pltpu.VMEM((NB, C), jnp.float32),
          pltpu.VMEM_SHARED((N, C), jnp.float32),
          pltpu.SemaphoreType.DMA,
          pltpu.SemaphoreType.DMA,
      ],
  )
  def body(xt_hbm, idxw_hbm, out_hbm, idx_t, acc0, acc1,
           tbl_s, sem_a, sem_b):
    sid = lax.axis_index("s")
    wid = sid * NUM_CORES + lax.axis_index("c")
    base = wid * PER_WORKER

    @pl.when(sid < 15)
    def _stage_main():
      off = pl.multiple_of(sid * STAGE_ROWS, 8)
      pltpu.sync_copy(xt_hbm.at[pl.ds(off, STAGE_ROWS)],
                      tbl_s.at[pl.ds(off, STAGE_ROWS)])

    @pl.when(sid == 15)
    def _stage_last():
      pltpu.sync_copy(xt_hbm.at[pl.ds(15 * STAGE_ROWS, STAGE_LAST)],
                      tbl_s.at[pl.ds(15 * STAGE_ROWS, STAGE_LAST)])

    pltpu.sync_copy(idxw_hbm.at[wid], idx_t)

    plsc.subcore_barrier()

    zv = jnp.zeros((LANES,), jnp.float32)
    accs = (acc0, acc1)
    sems = (sem_a, sem_b)

    def zero(acc):
      @pl.loop(0, NB)
      def _z(r):
        for cs in range(C // LANES):
          acc[r, pl.ds(cs * LANES, LANES)] = zv

    def fire(c, acc, sem):
      @pl.loop(0, K)
      def _f(k):
        pltpu.async_copy(tbl_s.at[idx_t.at[c * K + k]], acc, sem, add=True)

    def drain(acc, sem):
      @pl.loop(0, K)
      def _d(k):
        pltpu.make_async_copy(tbl_s.at[idx_t.at[0]], acc, sem).wait()

    for c in range(NCHUNKS):
      b, ob = c % 2, (c - 1) % 2
      zero(accs[b])
      fire(c, accs[b], sems[b])
      if c > 0:
        drain(accs[ob], sems[ob])
        pltpu.sync_copy(accs[ob], out_hbm.at[pl.ds(base + (c - 1) * NB, NB)])
    last = NCHUNKS - 1
    drain(accs[last % 2], sems[last % 2])
    pltpu.sync_copy(accs[last % 2],
                    out_hbm.at[pl.ds(base + last * NB, NB)])

  return body(xt, idx_w)


BN = 2048


def _tc_self(x2d, w, b1d, eps2d):
  def body(eps_ref, w_ref, b_ref, x_ref, o_ref):
    scale = 1.0 + eps_ref[0, 0]
    o_ref[...] = lax.dot_general(
        x_ref[...] * scale, w_ref[...],
        dimension_numbers=(((0,), (1,)), ((), ())),
        preferred_element_type=jnp.float32,
    ) + b_ref[...]

  grid = (pl.cdiv(N, BN),)
  return pl.pallas_call(
      body,
      grid=grid,
      in_specs=[
          pl.BlockSpec((1, 1), lambda i: (0, 0)),
          pl.BlockSpec((C, C), lambda i: (0, 0)),
          pl.BlockSpec((1, C), lambda i: (0, 0)),
          pl.BlockSpec((C, BN), lambda i: (0, i)),
      ],
      out_specs=pl.BlockSpec((BN, C), lambda i: (i, 0)),
      out_shape=jax.ShapeDtypeStruct((N, C), jnp.float32),
  )(eps2d, w, b1d, x2d)


def _tc_neigh(t1t, s, w):
  def body(w_ref, t1_ref, s_ref, o_ref):
    t2 = lax.dot_general(
        s_ref[...], w_ref[...],
        dimension_numbers=(((1,), (1,)), ((), ())),
        preferred_element_type=jnp.float32,
    )
    o_ref[...] = jnp.maximum(t1_ref[...] + t2, 0.0)

  grid = (pl.cdiv(N, BN),)
  return pl.pallas_call(
      body,
      grid=grid,
      in_specs=[
          pl.BlockSpec((C, C), lambda i: (0, 0)),
          pl.BlockSpec((BN, C), lambda i: (i, 0)),
          pl.BlockSpec((BN, C), lambda i: (i, 0)),
      ],
      out_specs=pl.BlockSpec((BN, C), lambda i: (i, 0)),
      out_shape=jax.ShapeDtypeStruct((N, C), jnp.float32),
  )(w, t1t, s)


def kernel(x, edge_index, W, b, eps):
  x2d = x.reshape(C, N)
  xt = x2d.T
  idx = edge_index[0].reshape(N, K)
  pad_idx = (jnp.arange((N_PAD - N) * K, dtype=jnp.int32) % N).reshape(
      N_PAD - N, K)
  idx_w = (
      jnp.concatenate([idx, pad_idx], axis=0)
      .reshape(NUM_WORKERS, NCHUNKS, NB, K)
      .transpose(0, 1, 3, 2)
      .reshape(NUM_WORKERS, NCHUNKS * K, NB)
  )
  s = _sc_gather_sum(xt, idx_w)

  b1d = b.reshape(1, C)
  eps2d = eps.reshape(1, 1)
  t1t = _tc_self(x2d, W, b1d, eps2d)
  out_t = _tc_neigh(t1t, s, W)
  return out_t.T.reshape(1, C, N, 1)
